# baseline (device time: 44582 ns/iter reference)
import jax
import jax.numpy as jnp
from jax import lax
from jax.experimental import pallas as pl
from jax.experimental.pallas import tpu as pltpu

N_DEV = 16
M = 512
N = 512
M_PER = M // N_DEV


def kernel(A, B):
    def body(a_ref, b_ref, out_ref, p_ref, comm_ref, send_sems, recv_sems):
        my = lax.axis_index("i")
        left = (my - 1 + N_DEV) % N_DEV
        right = (my + 1) % N_DEV

        barrier_sem = pltpu.get_barrier_semaphore()
        for nbr in (left, right):
            pl.semaphore_signal(
                barrier_sem, inc=1,
                device_id=(nbr,), device_id_type=pl.DeviceIdType.MESH,
            )
        pl.semaphore_wait(barrier_sem, 2)

        p_ref[:, :] = jnp.dot(
            a_ref[:, :].astype(jnp.bfloat16),
            b_ref[:, :].astype(jnp.bfloat16),
            preferred_element_type=jnp.float32,
        )

        c0 = (my - 1 + N_DEV) % N_DEV
        comm_ref[0, :, :] = p_ref[pl.ds(c0 * M_PER, M_PER), :]

        for s in range(N_DEV - 1):
            rdma = pltpu.make_async_remote_copy(
                src_ref=comm_ref.at[s],
                dst_ref=comm_ref.at[s + 1],
                send_sem=send_sems.at[s],
                recv_sem=recv_sems.at[s],
                device_id=(right,),
                device_id_type=pl.DeviceIdType.MESH,
            )
            rdma.start()
            rdma.wait()
            c = (my - 2 - s + 2 * N_DEV) % N_DEV
            comm_ref[s + 1, :, :] = (
                comm_ref[s + 1, :, :] + p_ref[pl.ds(c * M_PER, M_PER), :]
            )

        out_ref[:, :] = comm_ref[N_DEV - 1, :, :]

    return pl.pallas_call(
        body,
        out_shape=jax.ShapeDtypeStruct((M_PER, N), jnp.float32),
        in_specs=[
            pl.BlockSpec(memory_space=pltpu.VMEM),
            pl.BlockSpec(memory_space=pltpu.VMEM),
        ],
        out_specs=pl.BlockSpec(memory_space=pltpu.VMEM),
        scratch_shapes=[
            pltpu.VMEM((M, N), jnp.float32),
            pltpu.VMEM((N_DEV, M_PER, N), jnp.float32),
            pltpu.SemaphoreType.DMA((N_DEV - 1,)),
            pltpu.SemaphoreType.DMA((N_DEV - 1,)),
        ],
        compiler_params=pltpu.CompilerParams(collective_id=0),
    )(A, B)


# device time: 14529 ns/iter; 3.0685x vs baseline; 3.0685x over previous
import functools

import jax
import jax.numpy as jnp
from jax import lax
from jax.experimental import pallas as pl
from jax.experimental.pallas import tpu as pltpu

N_DEV = 16
M = 512
N = 512
M_PER = M // N_DEV


def kernel(A, B):
    def body(a_ref, b_ref, out_ref, p_ref, stage_ref, comm_ref,
             send_sems, recv_sems):
        my = lax.axis_index("i")

        barrier_sem = pltpu.get_barrier_semaphore()
        for k in range(1, N_DEV):
            pl.semaphore_signal(
                barrier_sem, inc=1,
                device_id=((my + k) % N_DEV,),
                device_id_type=pl.DeviceIdType.MESH,
            )
        pl.semaphore_wait(barrier_sem, N_DEV - 1)

        p_ref[:, :] = jnp.dot(
            a_ref[:, :].astype(jnp.bfloat16),
            b_ref[:, :].astype(jnp.bfloat16),
            preferred_element_type=jnp.float32,
        )
        stage_ref[:, :] = p_ref[:, :].astype(jnp.bfloat16)

        sends = []
        for k in range(1, N_DEV):
            t = (my + k) % N_DEV
            rdma = pltpu.make_async_remote_copy(
                src_ref=stage_ref.at[pl.ds(t * M_PER, M_PER), :],
                dst_ref=comm_ref.at[my],
                send_sem=send_sems.at[k - 1],
                recv_sem=recv_sems.at[my],
                device_id=(t,),
                device_id_type=pl.DeviceIdType.MESH,
            )
            rdma.start()
            sends.append(rdma)

        out_ref[:, :] = p_ref[pl.ds(my * M_PER, M_PER), :]

        for k in range(1, N_DEV):
            s = (my + k) % N_DEV
            recv = pltpu.make_async_remote_copy(
                src_ref=stage_ref.at[pl.ds(0, M_PER), :],
                dst_ref=comm_ref.at[s],
                send_sem=send_sems.at[0],
                recv_sem=recv_sems.at[s],
                device_id=(s,),
                device_id_type=pl.DeviceIdType.MESH,
            )
            recv.wait_recv()
            out_ref[:, :] = out_ref[:, :] + comm_ref[s].astype(jnp.float32)

        for rdma in sends:
            rdma.wait_send()

    return pl.pallas_call(
        body,
        out_shape=jax.ShapeDtypeStruct((M_PER, N), jnp.float32),
        in_specs=[
            pl.BlockSpec(memory_space=pltpu.VMEM),
            pl.BlockSpec(memory_space=pltpu.VMEM),
        ],
        out_specs=pl.BlockSpec(memory_space=pltpu.VMEM),
        scratch_shapes=[
            pltpu.VMEM((M, N), jnp.float32),
            pltpu.VMEM((M, N), jnp.bfloat16),
            pltpu.VMEM((N_DEV, M_PER, N), jnp.bfloat16),
            pltpu.SemaphoreType.DMA((N_DEV - 1,)),
            pltpu.SemaphoreType.DMA((N_DEV,)),
        ],
        compiler_params=pltpu.CompilerParams(collective_id=0),
    )(A, B)


# device time: 14137 ns/iter; 3.1536x vs baseline; 1.0277x over previous
import jax
import jax.numpy as jnp
from jax import lax
from jax.experimental import pallas as pl
from jax.experimental.pallas import tpu as pltpu

N_DEV = 16
M = 512
N = 512
M_PER = M // N_DEV


def kernel(A, B):
    def body(a_ref, b_ref, out_ref, stage_ref, comm_ref,
             send_sems, recv_sems):
        my = lax.axis_index("i")

        barrier_sem = pltpu.get_barrier_semaphore()
        for k in range(1, N_DEV):
            pl.semaphore_signal(
                barrier_sem, inc=1,
                device_id=((my + k) % N_DEV,),
                device_id_type=pl.DeviceIdType.MESH,
            )

        stage_ref[:, :] = jnp.dot(
            a_ref[:, :].astype(jnp.bfloat16),
            b_ref[:, :].astype(jnp.bfloat16),
            preferred_element_type=jnp.float32,
        ).astype(jnp.bfloat16)

        comm_ref[my, :, :] = stage_ref[pl.ds(my * M_PER, M_PER), :]

        pl.semaphore_wait(barrier_sem, N_DEV - 1)

        sends = []
        for k in range(1, N_DEV):
            t = (my + k) % N_DEV
            rdma = pltpu.make_async_remote_copy(
                src_ref=stage_ref.at[pl.ds(t * M_PER, M_PER), :],
                dst_ref=comm_ref.at[my],
                send_sem=send_sems.at[k - 1],
                recv_sem=recv_sems.at[my],
                device_id=(t,),
                device_id_type=pl.DeviceIdType.MESH,
            )
            rdma.start()
            sends.append(rdma)

        for k in range(1, N_DEV):
            s = (my + k) % N_DEV
            recv = pltpu.make_async_remote_copy(
                src_ref=stage_ref.at[pl.ds(0, M_PER), :],
                dst_ref=comm_ref.at[s],
                send_sem=send_sems.at[0],
                recv_sem=recv_sems.at[s],
                device_id=(s,),
                device_id_type=pl.DeviceIdType.MESH,
            )
            recv.wait_recv()

        out_ref[:, :] = jnp.sum(
            comm_ref[:, :, :].astype(jnp.float32), axis=0
        )

        for rdma in sends:
            rdma.wait_send()

    return pl.pallas_call(
        body,
        out_shape=jax.ShapeDtypeStruct((M_PER, N), jnp.float32),
        in_specs=[
            pl.BlockSpec(memory_space=pltpu.VMEM),
            pl.BlockSpec(memory_space=pltpu.VMEM),
        ],
        out_specs=pl.BlockSpec(memory_space=pltpu.VMEM),
        scratch_shapes=[
            pltpu.VMEM((M, N), jnp.bfloat16),
            pltpu.VMEM((N_DEV, M_PER, N), jnp.bfloat16),
            pltpu.SemaphoreType.DMA((N_DEV - 1,)),
            pltpu.SemaphoreType.DMA((N_DEV,)),
        ],
        compiler_params=pltpu.CompilerParams(collective_id=0),
    )(A, B)


# device time: 12211 ns/iter; 3.6510x vs baseline; 1.1577x over previous
import jax
import jax.numpy as jnp
from jax import lax
from jax.experimental import pallas as pl
from jax.experimental.pallas import tpu as pltpu

N_DEV = 16
M = 512
N = 512
M_PER = M // N_DEV


def kernel(A, B):
    def body(a_ref, b_ref, out_ref, stage_ref, comm_ref,
             send_sems, recv_sems, ready_sems):
        my = lax.axis_index("i")

        barrier_sem = pltpu.get_barrier_semaphore()
        pl.semaphore_signal(barrier_sem, inc=1)
        pl.semaphore_wait(barrier_sem, 1)

        for k in range(1, N_DEV):
            pl.semaphore_signal(
                ready_sems.at[my], inc=1,
                device_id=((my + k) % N_DEV,),
                device_id_type=pl.DeviceIdType.MESH,
            )

        stage_ref[:, :] = jnp.dot(
            a_ref[:, :].astype(jnp.bfloat16),
            b_ref[:, :].astype(jnp.bfloat16),
            preferred_element_type=jnp.float32,
        ).astype(jnp.bfloat16)

        comm_ref[my, :, :] = stage_ref[pl.ds(my * M_PER, M_PER), :]

        for t in range(N_DEV):
            @pl.when(t != my)
            def _(t=t):
                pl.semaphore_wait(ready_sems.at[t], 1)
                rdma = pltpu.make_async_remote_copy(
                    src_ref=stage_ref.at[pl.ds(t * M_PER, M_PER), :],
                    dst_ref=comm_ref.at[my],
                    send_sem=send_sems.at[t],
                    recv_sem=recv_sems.at[my],
                    device_id=(t,),
                    device_id_type=pl.DeviceIdType.MESH,
                )
                rdma.start()

        for s in range(N_DEV):
            @pl.when(s != my)
            def _(s=s):
                recv = pltpu.make_async_remote_copy(
                    src_ref=stage_ref.at[pl.ds(0, M_PER), :],
                    dst_ref=comm_ref.at[s],
                    send_sem=send_sems.at[my],
                    recv_sem=recv_sems.at[s],
                    device_id=(s,),
                    device_id_type=pl.DeviceIdType.MESH,
                )
                recv.wait_recv()

        out_ref[:, :] = jnp.sum(
            comm_ref[:, :, :].astype(jnp.float32), axis=0
        )

        for t in range(N_DEV):
            @pl.when(t != my)
            def _(t=t):
                rdma = pltpu.make_async_remote_copy(
                    src_ref=stage_ref.at[pl.ds(t * M_PER, M_PER), :],
                    dst_ref=comm_ref.at[my],
                    send_sem=send_sems.at[t],
                    recv_sem=recv_sems.at[my],
                    device_id=(t,),
                    device_id_type=pl.DeviceIdType.MESH,
                )
                rdma.wait_send()

    return pl.pallas_call(
        body,
        out_shape=jax.ShapeDtypeStruct((M_PER, N), jnp.float32),
        in_specs=[
            pl.BlockSpec(memory_space=pltpu.VMEM),
            pl.BlockSpec(memory_space=pltpu.VMEM),
        ],
        out_specs=pl.BlockSpec(memory_space=pltpu.VMEM),
        scratch_shapes=[
            pltpu.VMEM((M, N), jnp.bfloat16),
            pltpu.VMEM((N_DEV, M_PER, N), jnp.bfloat16),
            pltpu.SemaphoreType.DMA((N_DEV,)),
            pltpu.SemaphoreType.DMA((N_DEV,)),
            pltpu.SemaphoreType.REGULAR((N_DEV,)),
        ],
        compiler_params=pltpu.CompilerParams(collective_id=0),
    )(A, B)
